# Initial kernel scaffold; baseline (speedup 1.0000x reference)
import functools

import jax
import jax.numpy as jnp
import numpy as np
from jax import lax
from jax.experimental import pallas as pl
from jax.experimental.pallas import tpu as pltpu
from jax.experimental.pallas import tpu_sc as plsc

N = 10000
NP = 10240
E = 320000
H = 128
L = 3
S4 = 4
WA = 144
WB = 16
NC, NS = 2, 16
NW = NC * NS
CH = 128
NCHUNK = 80
EP = NW * NCHUNK * CH
RPS = NP // NS
PAD_ROW = N + 100
BLK = 1024
NBLK = NP // BLK
KS = [5000, 2500, 1250]
F32 = jnp.float32
NEG_INF = float("-inf")


def _edge_agg(table, src, dst, zeros, width):
    mesh = plsc.VectorSubcoreMesh(core_axis_name="c", subcore_axis_name="s",
                                  num_cores=NC, num_subcores=NS)

    @functools.partial(
        pl.kernel,
        out_type=jax.ShapeDtypeStruct((NC * NP, width), F32),
        mesh=mesh,
        scratch_types=[
            pltpu.VMEM((CH,), jnp.int32),
            pltpu.VMEM((CH,), jnp.int32),
            pltpu.VMEM((CH, width), F32),
            pltpu.VMEM_SHARED((NP, width), F32),
            pltpu.SemaphoreType.DMA,
        ],
    )
    def body(tab_hbm, src_hbm, dst_hbm, zero_hbm, out_hbm,
             srcv, dstv, rows, acc, sem):
        c = lax.axis_index("c")
        s = lax.axis_index("s")
        wid = s * NC + c
        pltpu.sync_copy(zero_hbm, acc.at[pl.ds(s * RPS, RPS)])
        plsc.subcore_barrier()

        def step(i, carry):
            base = wid * (NCHUNK * CH) + i * CH
            pltpu.sync_copy(src_hbm.at[pl.ds(base, CH)], srcv)
            pltpu.sync_copy(dst_hbm.at[pl.ds(base, CH)], dstv)
            pltpu.async_copy(tab_hbm.at[srcv], rows, sem).wait()
            pltpu.sync_copy(rows, acc.at[dstv], add=True)
            return carry

        lax.fori_loop(0, NCHUNK, step, 0)
        plsc.subcore_barrier()
        pltpu.sync_copy(acc.at[pl.ds(s * RPS, RPS)],
                        out_hbm.at[pl.ds(c * NP + s * RPS, RPS)])

    return body(table, src, dst, zeros)


def _conv_body(S_ref, tab_ref, Wl_ref, bl_ref, Wr_ref, wrel_ref, wroot_ref,
               hn_ref, srt_ref):
    Ssum = S_ref[0] + S_ref[1]
    h = tab_ref[:, :H]
    alive = tab_ref[:, H:H + 1]
    deg = Ssum[:, H:H + 1]
    mean_nb = Ssum[:, :H] / jnp.maximum(deg, 1.0)
    hn = jax.nn.relu(
        jnp.dot(mean_nb, Wl_ref[...], preferred_element_type=F32)
        + bl_ref[...]
        + jnp.dot(h, Wr_ref[...], preferred_element_type=F32))
    hn_ref[...] = hn
    sr = jnp.sum(hn * wrel_ref[...], axis=1, keepdims=True) * alive
    sv = jnp.sum(hn * wroot_ref[...], axis=1, keepdims=True)
    srt_ref[...] = jnp.concatenate(
        [sr, sv, jnp.zeros((BLK, WB - 2), F32)], axis=1)


def _conv(Spart, table, Wl, bl, Wr, wrel, wroot):
    return pl.pallas_call(
        _conv_body,
        grid=(NBLK,),
        in_specs=[
            pl.BlockSpec((NC, BLK, WA), lambda i: (0, i, 0)),
            pl.BlockSpec((BLK, WA), lambda i: (i, 0)),
            pl.BlockSpec((H, H), lambda i: (0, 0)),
            pl.BlockSpec((1, H), lambda i: (0, 0)),
            pl.BlockSpec((H, H), lambda i: (0, 0)),
            pl.BlockSpec((1, H), lambda i: (0, 0)),
            pl.BlockSpec((1, H), lambda i: (0, 0)),
        ],
        out_specs=[
            pl.BlockSpec((BLK, H), lambda i: (i, 0)),
            pl.BlockSpec((BLK, WB), lambda i: (i, 0)),
        ],
        out_shape=[
            jax.ShapeDtypeStruct((NP, H), F32),
            jax.ShapeDtypeStruct((NP, WB), F32),
        ],
    )(Spart, table, Wl, bl, Wr, wrel, wroot)


def _topk_body(k, aggs_ref, sv_ref, alive_ref, brel_ref, sel_ref, gate_ref):
    aggs = aggs_ref[0] + aggs_ref[1]
    score = jnp.tanh(aggs + brel_ref[0, 0] + sv_ref[...])
    alive = alive_ref[...] > 0.0
    bi = lax.bitcast_convert_type(score, jnp.int32)
    ki = jnp.where(bi < 0, bi ^ jnp.int32(0x7FFFFFFF), bi)
    imin = jnp.int32(-2147483648)
    ki = jnp.where(alive, ki, imin)

    def count_ge(t):
        return jnp.sum((ki >= t).astype(jnp.int32))

    t0 = jnp.where(count_ge(jnp.int32(0)) >= k, jnp.int32(0), imin)

    def bit_step(i, t):
        cand = t | (jnp.int32(1) << (jnp.int32(30) - i))
        return jnp.where(count_ge(cand) >= k, cand, t)

    T = lax.fori_loop(0, 31, bit_step, t0)
    g = jnp.sum((ki > T).astype(jnp.int32))
    tie = (ki == T)
    m = (k - g).astype(F32)
    tie_f = tie.astype(F32)
    r_i = lax.broadcasted_iota(jnp.int32, (128, 128), 0)
    c_i = lax.broadcasted_iota(jnp.int32, (128, 128), 1)
    U = (r_i < c_i).astype(F32)
    within = jnp.dot(tie_f, U, preferred_element_type=F32)
    rowsum = jnp.sum(tie_f, axis=1)
    rowsum_p = jnp.concatenate(
        [rowsum.reshape(1, 80), jnp.zeros((1, 48), F32)], axis=1)
    rowoff = jnp.dot(rowsum_p, U, preferred_element_type=F32)
    rank = within + rowoff[0, :80].reshape(80, 1)
    sel = (ki > T) | (tie & (rank < m))
    sel_ref[...] = sel.astype(F32)
    gate_ref[...] = jnp.where(sel, score, 0.0)


def _topk(k, aggs2, sv2, alive2, brel):
    return pl.pallas_call(
        functools.partial(_topk_body, k),
        grid=(1,),
        in_specs=[
            pl.BlockSpec((NC, 80, 128), lambda i: (0, 0, 0)),
            pl.BlockSpec((80, 128), lambda i: (0, 0)),
            pl.BlockSpec((80, 128), lambda i: (0, 0)),
            pl.BlockSpec((1, 1), lambda i: (0, 0)),
        ],
        out_specs=[
            pl.BlockSpec((80, 128), lambda i: (0, 0)),
            pl.BlockSpec((80, 128), lambda i: (0, 0)),
        ],
        out_shape=[
            jax.ShapeDtypeStruct((80, 128), F32),
            jax.ShapeDtypeStruct((80, 128), F32),
        ],
    )(aggs2, sv2, alive2, brel)


def _pool_body(hn_ref, sel_ref, gate_ref, oneh_ref,
               tabn_ref, stats_ref, fmax_ref):
    hg = hn_ref[...] * gate_ref[...]
    tabn = jnp.concatenate(
        [hg, sel_ref[...], gate_ref[...], jnp.zeros((BLK, WA - H - 2), F32)],
        axis=1)
    tabn_ref[...] = tabn
    oneh = oneh_ref[...]
    stats_part = lax.dot_general(oneh, tabn, (((0,), (0,)), ((), ())),
                                 preferred_element_type=F32)

    @pl.when(pl.program_id(0) == 0)
    def _init():
        stats_ref[...] = jnp.zeros((8, WA), F32)
        fmax_ref[...] = jnp.full((8, H), NEG_INF, F32)

    stats_ref[...] += stats_part
    for s in range(S4):
        msk = sel_ref[...] * oneh[:, s:s + 1]
        cand = jnp.where(msk > 0.0, hg, NEG_INF)
        fmax_ref[s:s + 1, :] = jnp.maximum(
            fmax_ref[s:s + 1, :], jnp.max(cand, axis=0, keepdims=True))


def _pool(hn, sel_col, gate_col, oneh):
    return pl.pallas_call(
        _pool_body,
        grid=(NBLK,),
        in_specs=[
            pl.BlockSpec((BLK, H), lambda i: (i, 0)),
            pl.BlockSpec((BLK, 1), lambda i: (i, 0)),
            pl.BlockSpec((BLK, 1), lambda i: (i, 0)),
            pl.BlockSpec((BLK, 8), lambda i: (i, 0)),
        ],
        out_specs=[
            pl.BlockSpec((BLK, WA), lambda i: (i, 0)),
            pl.BlockSpec((8, WA), lambda i: (0, 0)),
            pl.BlockSpec((8, H), lambda i: (0, 0)),
        ],
        out_shape=[
            jax.ShapeDtypeStruct((NP, WA), F32),
            jax.ShapeDtypeStruct((8, WA), F32),
            jax.ShapeDtypeStruct((8, H), F32),
        ],
    )(hn, sel_col, gate_col, oneh)


def _tail_body(s1, s2, s3, f1, f2, f3, WvT, bv, WoT, bo,
               g1, b1, g2, b2, Wc1T, bc1, Wc2Tp, bc2p, out_ref):
    embs = []
    for st_ref, fm_ref in ((s1, f1), (s2, f2), (s3, f3)):
        st = st_ref[...]
        cnt = st[:, H:H + 1]
        mssum = st[:, H + 1:H + 2]
        present = cnt > 0.0
        ms = jnp.where(present, mssum / jnp.maximum(cnt, 1.0), 0.0)
        w = jnp.where(present, ms / jnp.sum(ms), 0.0)
        fmean = st[:, :H] / jnp.maximum(cnt, 1.0)
        fmaxw = jnp.where(present, fm_ref[...], 0.0)
        embs.append(jnp.sum(w * fmean, axis=0, keepdims=True))
        embs.append(jnp.sum(w * fmaxw, axis=0, keepdims=True))
    xcat = jnp.concatenate(embs, axis=1)

    def layernorm(x, g, b):
        mu = jnp.mean(x, axis=1, keepdims=True)
        var = jnp.mean((x - mu) ** 2, axis=1, keepdims=True)
        return (x - mu) / jnp.sqrt(var + 1e-5) * g + b

    xn = layernorm(xcat, g1[...], b1[...])
    v = jnp.dot(xn, WvT[...], preferred_element_type=F32) + bv[...]
    attn = jnp.dot(v, WoT[...], preferred_element_type=F32) + bo[...]
    y = layernorm(attn + xn, g2[...], b2[...])
    la0 = jnp.sum(y[:, :256])
    la1 = jnp.sum(y[:, 256:512])
    la2 = jnp.sum(y[:, 512:768])
    lamin = jnp.minimum(la0, jnp.minimum(la1, la2))
    la0, la1, la2 = la0 - lamin + 1e-8, la1 - lamin + 1e-8, la2 - lamin + 1e-8
    lasum = la0 + la1 + la2
    io = lax.broadcasted_iota(jnp.int32, (1, 128), 1)
    la_row = (jnp.where(io == 0, la0, 0.0) + jnp.where(io == 1, la1, 0.0)
              + jnp.where(io == 2, la2, 0.0)) / lasum
    la_row = jnp.where(io < 3, la_row, 0.0)
    z = jax.nn.relu(y)
    hid = jnp.dot(z, Wc1T[...], preferred_element_type=F32) + bc1[...]
    logits = jnp.dot(hid, Wc2Tp[...], preferred_element_type=F32) + bc2p[...]
    l0 = jnp.sum(jnp.where(io == 0, logits, 0.0))
    l1 = jnp.sum(jnp.where(io == 1, logits, 0.0))
    mx = jnp.maximum(l0, l1)
    e0 = jnp.exp(l0 - mx)
    e1 = jnp.exp(l1 - mx)
    p_row = (jnp.where(io == 0, e0, 0.0) + jnp.where(io == 1, e1, 0.0)) \
        / (e0 + e1)
    out_ref[...] = jnp.concatenate(
        [logits, p_row, la_row, jnp.zeros((5, 128), F32)], axis=0)


def _tail(stats, fmaxs, WvT, bv, WoT, bo, g1, b1, g2, b2,
          Wc1T, bc1, Wc2Tp, bc2p):
    args = [stats[0], stats[1], stats[2], fmaxs[0], fmaxs[1], fmaxs[2],
            WvT, bv, WoT, bo, g1, b1, g2, b2, Wc1T, bc1, Wc2Tp, bc2p]
    specs = [pl.BlockSpec(a.shape, lambda i: tuple(0 for _ in range(a.ndim)))
             for a in args]
    return pl.pallas_call(
        _tail_body,
        grid=(1,),
        in_specs=specs,
        out_specs=pl.BlockSpec((8, 128), lambda i: (0, 0)),
        out_shape=jax.ShapeDtypeStruct((8, 128), F32),
    )(*args)


def kernel(x, edge_index, node_attr, batch, label, sage_Wl, sage_bl, sage_Wr,
           pool_Wrel, pool_brel, pool_Wroot, ln1_g, ln1_b, Wqkv, bqkv, Wo, bo,
           ln2_g, ln2_b, Wc1, bc1, Wc2, bc2):
    D = 2 * H * L
    src = edge_index[0].astype(jnp.int32)
    dst = edge_index[1].astype(jnp.int32)
    src_p = jnp.concatenate(
        [src, jnp.full((EP - E,), PAD_ROW, jnp.int32)])
    dst_p = jnp.concatenate(
        [dst, jnp.full((EP - E,), PAD_ROW, jnp.int32)])

    row_ids = jnp.arange(NP, dtype=jnp.int32)
    alive_col = (row_ids < N).astype(F32).reshape(NP, 1)
    x_pad = jnp.concatenate([x, jnp.zeros((NP - N, H), F32)], axis=0)
    table = jnp.concatenate(
        [x_pad * alive_col, alive_col, jnp.zeros((NP, WA - H - 1), F32)],
        axis=1)
    alive2 = alive_col.reshape(80, 128)

    na_pad = jnp.concatenate(
        [node_attr.astype(jnp.int32), jnp.full((NP - N,), S4 + 3, jnp.int32)])
    oneh = (na_pad[:, None] == jnp.arange(S4, dtype=jnp.int32)[None, :])
    oneh = jnp.concatenate(
        [oneh.astype(F32), jnp.zeros((NP, 4), F32)], axis=1)

    zerosA = jnp.zeros((RPS, WA), F32)
    zerosB = jnp.zeros((RPS, WB), F32)

    stats_l, fmax_l = [], []
    for i in range(L):
        Spart = _edge_agg(table, src_p, dst_p, zerosA, WA).reshape(NC, NP, WA)
        hn, srt = _conv(
            Spart, table, sage_Wl[i], sage_bl[i].reshape(1, H), sage_Wr[i],
            pool_Wrel[i].reshape(1, H), pool_Wroot[i].reshape(1, H))
        aggsT = _edge_agg(srt, src_p, dst_p, zerosB, WB).reshape(NC, NP, WB)
        aggs2 = aggsT[:, :, 0].reshape(NC, 80, 128)
        sv2 = srt[:, 1].reshape(80, 128)
        sel2, gate2 = _topk(KS[i], aggs2, sv2, alive2,
                            pool_brel[i].reshape(1, 1))
        table, stats, fmax = _pool(hn, sel2.reshape(NP, 1),
                                   gate2.reshape(NP, 1), oneh)
        stats_l.append(stats)
        fmax_l.append(fmax)
        alive2 = sel2

    Wv = Wqkv[2 * D:3 * D]
    bv = bqkv[2 * D:3 * D]
    out8 = _tail(
        stats_l, fmax_l,
        Wv.T, bv.reshape(1, D), Wo.T, bo.reshape(1, D),
        ln1_g.reshape(1, D), ln1_b.reshape(1, D),
        ln2_g.reshape(1, D), ln2_b.reshape(1, D),
        Wc1.T, bc1.reshape(1, D // 2),
        jnp.concatenate([Wc2.T, jnp.zeros((D // 2, 126), F32)], axis=1),
        jnp.concatenate([bc2, jnp.zeros((126,), F32)]).reshape(1, 128))
    logits = out8[0:1, 0:2]
    probs = out8[1:2, 0:2]
    la = out8[2, 0:3]
    return (logits, probs, la, label)


# speedup vs baseline: 1.2662x; 1.2662x over previous
import functools

import jax
import jax.numpy as jnp
import numpy as np
from jax import lax
from jax.experimental import pallas as pl
from jax.experimental.pallas import tpu as pltpu
from jax.experimental.pallas import tpu_sc as plsc

N = 10000
NP = 10240
E = 320000
H = 128
L = 3
S4 = 4
WA = 144
NC, NS = 2, 16
NW = NC * NS
OWN = NP // NW
ACC = OWN + 8
DUMP = OWN
CAP = 11264
NCHUNK = CAP // 128
PAD_ROW = N + 100
BLK = 1024
NBLK = NP // BLK
KS = [5000, 2500, 1250]
F32 = jnp.float32
NEG_INF = float("-inf")
E0 = np.zeros((16,), np.float32)
E0[0] = 1.0


def _edge_agg(table, srcb, dstb, zeros, in_width, out_width):
    mesh = plsc.VectorSubcoreMesh(core_axis_name="c", subcore_axis_name="s",
                                  num_cores=NC, num_subcores=NS)
    ncol = out_width // 16

    @functools.partial(
        pl.kernel,
        out_type=jax.ShapeDtypeStruct((NP, out_width), F32),
        mesh=mesh,
        scratch_types=[
            pltpu.VMEM((128,), jnp.int32),
            pltpu.VMEM((128,), jnp.int32),
            pltpu.VMEM((128, in_width), F32),
            pltpu.VMEM((ACC, out_width), F32),
            pltpu.SemaphoreType.DMA,
        ],
    )
    def body(tab_hbm, srcb_hbm, dstb_hbm, zero_hbm, out_hbm,
             srcv, dstv, rows, acc, sem):
        c = lax.axis_index("c")
        s = lax.axis_index("s")
        wid = s * NC + c
        pltpu.sync_copy(zero_hbm, acc)

        def step(i, carry):
            base = wid * CAP + i * 128
            pltpu.sync_copy(srcb_hbm.at[pl.ds(base, 128)], srcv)
            pltpu.sync_copy(dstb_hbm.at[pl.ds(base, 128)], dstv)
            pltpu.async_copy(tab_hbm.at[srcv], rows, sem).wait()

            def group16(g, carry2):
                dv = dstv[pl.ds(g * 16, 16)]
                for j in range(16):
                    d = dv[j]
                    e = g * 16 + j
                    for cc in range(ncol):
                        sl = pl.ds(cc * 16, 16)
                        acc[d, sl] = acc[d, sl] + rows[e, sl]
                return carry2

            lax.fori_loop(0, 8, group16, 0)
            return carry

        lax.fori_loop(0, NCHUNK, step, 0)
        pltpu.sync_copy(acc.at[pl.ds(0, OWN)],
                        out_hbm.at[pl.ds(wid * OWN, OWN)])

    return body(table, srcb, dstb, zeros)


def _conv_body(S_ref, tab_ref, alive_ref, Wl_ref, bl_ref, Wr_ref, hn_ref):
    Ssum = S_ref[...]
    h = tab_ref[:, :H]
    deg = Ssum[:, H:H + 1]
    mean_nb = Ssum[:, :H] / jnp.maximum(deg, 1.0)
    hn = jax.nn.relu(
        jnp.dot(mean_nb, Wl_ref[...], preferred_element_type=F32)
        + bl_ref[...]
        + jnp.dot(h, Wr_ref[...], preferred_element_type=F32))
    hn_ref[...] = hn * alive_ref[...]


def _conv(S, table, alive_col, Wl, bl, Wr):
    return pl.pallas_call(
        _conv_body,
        grid=(NBLK,),
        in_specs=[
            pl.BlockSpec((BLK, WA), lambda i: (i, 0)),
            pl.BlockSpec((BLK, 2 * H), lambda i: (i, 0)),
            pl.BlockSpec((BLK, 1), lambda i: (i, 0)),
            pl.BlockSpec((H, H), lambda i: (0, 0)),
            pl.BlockSpec((1, H), lambda i: (0, 0)),
            pl.BlockSpec((H, H), lambda i: (0, 0)),
        ],
        out_specs=pl.BlockSpec((BLK, H), lambda i: (i, 0)),
        out_shape=jax.ShapeDtypeStruct((NP, H), F32),
    )(S, table, alive_col, Wl, bl, Wr)


def _score_body(aggp_ref, hn_ref, wrel_ref, wroot_ref, brel_ref, out_ref):
    arg = (jnp.dot(aggp_ref[...], wrel_ref[...], preferred_element_type=F32)
           + brel_ref[0, 0]
           + jnp.dot(hn_ref[...], wroot_ref[...], preferred_element_type=F32))
    out_ref[...] = jnp.tanh(arg)


def _score(aggp, hn, wrel, wroot, brel):
    return pl.pallas_call(
        _score_body,
        grid=(NBLK,),
        in_specs=[
            pl.BlockSpec((BLK, H), lambda i: (i, 0)),
            pl.BlockSpec((BLK, H), lambda i: (i, 0)),
            pl.BlockSpec((H, 1), lambda i: (0, 0)),
            pl.BlockSpec((H, 1), lambda i: (0, 0)),
            pl.BlockSpec((1, 1), lambda i: (0, 0)),
        ],
        out_specs=pl.BlockSpec((BLK, 1), lambda i: (i, 0)),
        out_shape=jax.ShapeDtypeStruct((NP, 1), F32),
    )(aggp, hn, wrel, wroot, brel)


def _topk_body(k, score_ref, alive_ref, sel_ref, gate_ref):
    score = score_ref[...]
    alive = alive_ref[...] > 0.0
    bi = lax.bitcast_convert_type(score, jnp.int32)
    ki = jnp.where(bi < 0, bi ^ jnp.int32(0x7FFFFFFF), bi)
    imin = jnp.int32(-2147483648)
    ki = jnp.where(alive, ki, imin)

    def count_ge(t):
        return jnp.sum((ki >= t).astype(jnp.int32))

    t0 = jnp.where(count_ge(jnp.int32(0)) >= k, jnp.int32(0), imin)

    def bit_step(i, t):
        cand = t | (jnp.int32(1) << (jnp.int32(30) - i))
        return jnp.where(count_ge(cand) >= k, cand, t)

    T = lax.fori_loop(0, 31, bit_step, t0)
    g = jnp.sum((ki > T).astype(jnp.int32))
    tie = (ki == T)
    m = (k - g).astype(F32)
    tie_f = tie.astype(F32)
    r_i = lax.broadcasted_iota(jnp.int32, (128, 128), 0)
    c_i = lax.broadcasted_iota(jnp.int32, (128, 128), 1)
    U = (r_i < c_i).astype(F32)
    within = jnp.dot(tie_f, U, preferred_element_type=F32)
    rowsum_c = jnp.sum(tie_f, axis=1, keepdims=True)
    rr = lax.broadcasted_iota(jnp.int32, (80, 80), 0)
    cc = lax.broadcasted_iota(jnp.int32, (80, 80), 1)
    L2 = (cc < rr).astype(F32)
    rowoff_c = jnp.dot(L2, rowsum_c, preferred_element_type=F32)
    rank = within + rowoff_c
    sel = (ki > T) | (tie & (rank < m))
    sel_ref[...] = sel.astype(F32)
    gate_ref[...] = jnp.where(sel, score, 0.0)


def _topk(k, score2, alive2):
    return pl.pallas_call(
        functools.partial(_topk_body, k),
        grid=(1,),
        in_specs=[
            pl.BlockSpec((80, 128), lambda i: (0, 0)),
            pl.BlockSpec((80, 128), lambda i: (0, 0)),
        ],
        out_specs=[
            pl.BlockSpec((80, 128), lambda i: (0, 0)),
            pl.BlockSpec((80, 128), lambda i: (0, 0)),
        ],
        out_shape=[
            jax.ShapeDtypeStruct((80, 128), F32),
            jax.ShapeDtypeStruct((80, 128), F32),
        ],
    )(score2, alive2)


def _pool_body(hn_ref, sel_ref, gate_ref, oneh_ref,
               tabn_ref, stats_ref, fmax_ref):
    hg = hn_ref[...] * gate_ref[...]
    tabn_ref[...] = jnp.concatenate(
        [hg, sel_ref[...], jnp.zeros((BLK, 127), F32)], axis=1)
    t144 = jnp.concatenate(
        [hg, sel_ref[...], gate_ref[...], jnp.zeros((BLK, WA - H - 2), F32)],
        axis=1)
    oneh = oneh_ref[...]
    stats_part = lax.dot_general(oneh, t144, (((0,), (0,)), ((), ())),
                                 preferred_element_type=F32)

    @pl.when(pl.program_id(0) == 0)
    def _init():
        stats_ref[...] = jnp.zeros((8, WA), F32)
        fmax_ref[...] = jnp.full((8, H), NEG_INF, F32)

    stats_ref[...] += stats_part
    for s in range(S4):
        msk = sel_ref[...] * oneh[:, s:s + 1]
        cand = jnp.where(msk > 0.0, hg, NEG_INF)
        fmax_ref[s:s + 1, :] = jnp.maximum(
            fmax_ref[s:s + 1, :], jnp.max(cand, axis=0, keepdims=True))


def _pool(hn, sel_col, gate_col, oneh):
    return pl.pallas_call(
        _pool_body,
        grid=(NBLK,),
        in_specs=[
            pl.BlockSpec((BLK, H), lambda i: (i, 0)),
            pl.BlockSpec((BLK, 1), lambda i: (i, 0)),
            pl.BlockSpec((BLK, 1), lambda i: (i, 0)),
            pl.BlockSpec((BLK, 8), lambda i: (i, 0)),
        ],
        out_specs=[
            pl.BlockSpec((BLK, 2 * H), lambda i: (i, 0)),
            pl.BlockSpec((8, WA), lambda i: (0, 0)),
            pl.BlockSpec((8, H), lambda i: (0, 0)),
        ],
        out_shape=[
            jax.ShapeDtypeStruct((NP, 2 * H), F32),
            jax.ShapeDtypeStruct((8, WA), F32),
            jax.ShapeDtypeStruct((8, H), F32),
        ],
    )(hn, sel_col, gate_col, oneh)


def _tail_body(s1, s2, s3, f1, f2, f3, WvT, bv, WoT, bo,
               g1, b1, g2, b2, Wc1T, bc1, Wc2Tp, bc2p, out_ref):
    embs = []
    for st_ref, fm_ref in ((s1, f1), (s2, f2), (s3, f3)):
        st = st_ref[...]
        cnt = st[:, H:H + 1]
        mssum = st[:, H + 1:H + 2]
        present = cnt > 0.0
        ms = jnp.where(present, mssum / jnp.maximum(cnt, 1.0), 0.0)
        w = jnp.where(present, ms / jnp.sum(ms), 0.0)
        fmean = st[:, :H] / jnp.maximum(cnt, 1.0)
        fmaxw = jnp.where(present, fm_ref[...], 0.0)
        embs.append(jnp.sum(w * fmean, axis=0, keepdims=True))
        embs.append(jnp.sum(w * fmaxw, axis=0, keepdims=True))
    xcat = jnp.concatenate(embs, axis=1)

    def layernorm(x, g, b):
        mu = jnp.mean(x, axis=1, keepdims=True)
        var = jnp.mean((x - mu) ** 2, axis=1, keepdims=True)
        return (x - mu) / jnp.sqrt(var + 1e-5) * g + b

    xn = layernorm(xcat, g1[...], b1[...])
    v = jnp.dot(xn, WvT[...], preferred_element_type=F32) + bv[...]
    attn = jnp.dot(v, WoT[...], preferred_element_type=F32) + bo[...]
    y = layernorm(attn + xn, g2[...], b2[...])
    la0 = jnp.sum(y[:, :256])
    la1 = jnp.sum(y[:, 256:512])
    la2 = jnp.sum(y[:, 512:768])
    lamin = jnp.minimum(la0, jnp.minimum(la1, la2))
    la0, la1, la2 = la0 - lamin + 1e-8, la1 - lamin + 1e-8, la2 - lamin + 1e-8
    lasum = la0 + la1 + la2
    io = lax.broadcasted_iota(jnp.int32, (1, 128), 1)
    la_row = (jnp.where(io == 0, la0, 0.0) + jnp.where(io == 1, la1, 0.0)
              + jnp.where(io == 2, la2, 0.0)) / lasum
    la_row = jnp.where(io < 3, la_row, 0.0)
    z = jax.nn.relu(y)
    hid = jnp.dot(z, Wc1T[...], preferred_element_type=F32) + bc1[...]
    logits = jnp.dot(hid, Wc2Tp[...], preferred_element_type=F32) + bc2p[...]
    l0 = jnp.sum(jnp.where(io == 0, logits, 0.0))
    l1 = jnp.sum(jnp.where(io == 1, logits, 0.0))
    mx = jnp.maximum(l0, l1)
    e0 = jnp.exp(l0 - mx)
    e1 = jnp.exp(l1 - mx)
    p_row = (jnp.where(io == 0, e0, 0.0) + jnp.where(io == 1, e1, 0.0)) \
        / (e0 + e1)
    out_ref[...] = jnp.concatenate(
        [logits, p_row, la_row, jnp.zeros((5, 128), F32)], axis=0)


def _tail(stats, fmaxs, WvT, bv, WoT, bo, g1, b1, g2, b2,
          Wc1T, bc1, Wc2Tp, bc2p):
    args = [stats[0], stats[1], stats[2], fmaxs[0], fmaxs[1], fmaxs[2],
            WvT, bv, WoT, bo, g1, b1, g2, b2, Wc1T, bc1, Wc2Tp, bc2p]
    specs = [pl.BlockSpec(a.shape, functools.partial(
        lambda nd, i: (0,) * nd, a.ndim)) for a in args]
    return pl.pallas_call(
        _tail_body,
        grid=(1,),
        in_specs=specs,
        out_specs=pl.BlockSpec((8, 128), lambda i: (0, 0)),
        out_shape=jax.ShapeDtypeStruct((8, 128), F32),
    )(*args)


def _bucket(src, dst):
    owner = dst // OWN
    perm = jnp.argsort(owner, stable=True)
    so = owner[perm]
    ss = src[perm]
    sd = dst[perm]
    counts = jnp.bincount(owner, length=NW)
    starts = (jnp.cumsum(counts) - counts).astype(jnp.int32)
    within = jnp.arange(E, dtype=jnp.int32) - starts[so]
    pos = jnp.where(within < CAP, so * CAP + within, -1)
    srcb = jnp.full((NW * CAP,), PAD_ROW, jnp.int32).at[pos].set(
        ss, mode="drop", unique_indices=True)
    dstb = jnp.full((NW * CAP,), DUMP, jnp.int32).at[pos].set(
        (sd - so * OWN).astype(jnp.int32), mode="drop", unique_indices=True)
    return srcb, dstb


def kernel(x, edge_index, node_attr, batch, label, sage_Wl, sage_bl, sage_Wr,
           pool_Wrel, pool_brel, pool_Wroot, ln1_g, ln1_b, Wqkv, bqkv, Wo, bo,
           ln2_g, ln2_b, Wc1, bc1, Wc2, bc2):
    D = 2 * H * L
    src = edge_index[0].astype(jnp.int32)
    dst = edge_index[1].astype(jnp.int32)
    srcb, dstb = _bucket(src, dst)

    row_ids = jnp.arange(NP, dtype=jnp.int32)
    alive_col = (row_ids < N).astype(F32).reshape(NP, 1)
    x_pad = jnp.concatenate([x, jnp.zeros((NP - N, H), F32)], axis=0)
    table = jnp.concatenate(
        [x_pad * alive_col, alive_col, jnp.zeros((NP, 127), F32)], axis=1)
    alive2 = alive_col.reshape(80, 128)

    na_pad = jnp.concatenate(
        [node_attr.astype(jnp.int32), jnp.full((NP - N,), S4 + 3, jnp.int32)])
    oneh = (na_pad[:, None] == jnp.arange(S4, dtype=jnp.int32)[None, :])
    oneh = jnp.concatenate(
        [oneh.astype(F32), jnp.zeros((NP, 4), F32)], axis=1)

    zerosA = jnp.zeros((ACC, WA), F32)
    zerosB = jnp.zeros((ACC, H), F32)

    stats_l, fmax_l = [], []
    for i in range(L):
        S = _edge_agg(table, srcb, dstb, zerosA, 2 * H, WA)
        hn = _conv(S, table, alive2.reshape(NP, 1), sage_Wl[i],
                   sage_bl[i].reshape(1, H), sage_Wr[i])
        aggp = _edge_agg(hn, srcb, dstb, zerosB, H, H)
        sc = _score(aggp, hn, pool_Wrel[i], pool_Wroot[i],
                    pool_brel[i].reshape(1, 1))
        sel2, gate2 = _topk(KS[i], sc.reshape(80, 128), alive2)
        table, stats, fmax = _pool(hn, sel2.reshape(NP, 1),
                                   gate2.reshape(NP, 1), oneh)
        stats_l.append(stats)
        fmax_l.append(fmax)
        alive2 = sel2

    Wv = Wqkv[2 * D:3 * D]
    bv = bqkv[2 * D:3 * D]
    out8 = _tail(
        stats_l, fmax_l,
        Wv.T, bv.reshape(1, D), Wo.T, bo.reshape(1, D),
        ln1_g.reshape(1, D), ln1_b.reshape(1, D),
        ln2_g.reshape(1, D), ln2_b.reshape(1, D),
        Wc1.T, bc1.reshape(1, D // 2),
        jnp.concatenate([Wc2.T, jnp.zeros((D // 2, 126), F32)], axis=1),
        jnp.concatenate([bc2, jnp.zeros((126,), F32)]).reshape(1, 128))
    logits = out8[0:1, 0:2]
    probs = out8[1:2, 0:2]
    la = out8[2, 0:3]
    return (logits, probs, la, label)
